# baseline (speedup 1.0000x reference)
import functools

import jax
import jax.numpy as jnp
from jax import lax
from jax.experimental import pallas as pl
from jax.experimental.pallas import tpu as pltpu
from jax.experimental.pallas import tpu_sc as plsc

N = 100000
E = 1600000
IN_CH = 128
F = 40
FQ = 8
NG = 5
K = 10
ALPHA = 0.1

B = 128
GB = 1024
SPB = GB // B
IC = 2 * GB // B
NT = 16
EROWS = 12800
E_PAD = EROWS * B
RPT = EROWS // NT
NIC0 = RPT // IC
NIC2 = RPT // 2 // IC
ROWS_PT = 6256
NROWS = ROWS_PT * NT
SENT = N
CPY = 368
NCPY = ROWS_PT // CPY
NP = 102400
OST = NP

DCH = 16
DNCH = 25

RB = 2000
GRID = N // RB

_mesh = plsc.VectorSubcoreMesh(core_axis_name="c", subcore_axis_name="s")


@functools.partial(
    pl.kernel,
    out_type=jax.ShapeDtypeStruct((2 * NROWS,), jnp.float32),
    mesh=_mesh,
    scratch_types=[
        pltpu.VMEM_SHARED((NROWS,), jnp.float32),
        pltpu.VMEM((DCH, B), jnp.int32),
        pltpu.VMEM((B,), jnp.float32),
        pltpu.VMEM((ROWS_PT,), jnp.float32),
        pltpu.SemaphoreType.DMA,
    ],
    compiler_params=pltpu.CompilerParams(use_tc_tiling_on_sc=False),
)
def _deg_kernel(dst2, deg_out, acc, didx, ones, zbuf, ssem):
    c = lax.axis_index("c")
    s = lax.axis_index("s")
    for i in range(B // 16):
        ones[pl.ds(i * 16, 16)] = jnp.ones((16,), jnp.float32)
    for i in range(ROWS_PT // 16):
        zbuf[pl.ds(i * 16, 16)] = jnp.zeros((16,), jnp.float32)
    pltpu.sync_copy(zbuf, acc.at[pl.ds(s * ROWS_PT, ROWS_PT)])
    plsc.subcore_barrier()

    w = c * NT + s
    r0 = w * (DCH * DNCH)

    def chunk(k, carry):
        rb = r0 + k * DCH
        pltpu.sync_copy(dst2.at[pl.ds(rb, DCH)], didx)
        for j in range(DCH):
            pltpu.async_copy(ones, acc.at[didx.at[j]], ssem, add=True)
        for j in range(DCH):
            pltpu.make_async_copy(ones, acc.at[didx.at[j]], ssem).wait()
        return carry

    lax.fori_loop(0, DNCH, chunk, 0)
    plsc.subcore_barrier()

    pltpu.sync_copy(acc.at[pl.ds(s * ROWS_PT, ROWS_PT)], zbuf)
    pltpu.sync_copy(zbuf, deg_out.at[pl.ds(c * NROWS + s * ROWS_PT, ROWS_PT)])


def _prep_body(x_ref, w1_ref, b1_ref, w2_ref, b2_ref, d0_ref, d1_ref,
               *outs):
    xb = x_ref[...]
    h = lax.dot_general(xb, w1_ref[...], (((1,), (1,)), ((), ())),
                        preferred_element_type=jnp.float32)
    h = jnp.maximum(h + b1_ref[...], 0.0)
    y = lax.dot_general(h, w2_ref[...], (((1,), (1,)), ((), ())),
                        preferred_element_type=jnp.float32)
    y = y + b2_ref[...]
    deg = d0_ref[...] + d1_ref[...] + 1.0
    dinv = lax.rsqrt(deg)
    g0 = y * dinv
    t0 = ALPHA * g0
    u0 = ALPHA * y
    for q in range(NG):
        outs[q][...] = g0[:, q * FQ:(q + 1) * FQ]
        outs[NG + q][...] = t0[:, q * FQ:(q + 1) * FQ]
        outs[2 * NG + q][...] = u0[:, q * FQ:(q + 1) * FQ]
    outs[3 * NG][...] = (1.0 - ALPHA) * dinv * dinv
    outs[3 * NG + 1][...] = (1.0 - ALPHA) * dinv


_q_spec = pl.BlockSpec((RB, FQ), lambda i: (i, 0))
_q_shape = jax.ShapeDtypeStruct((N, FQ), jnp.float32)
_d_spec = pl.BlockSpec((RB, 1), lambda i: (i, 0))

_prep = pl.pallas_call(
    _prep_body,
    grid=(GRID,),
    in_specs=[
        pl.BlockSpec((RB, IN_CH), lambda i: (i, 0)),
        pl.BlockSpec((IN_CH, IN_CH), lambda i: (0, 0)),
        pl.BlockSpec((1, IN_CH), lambda i: (0, 0)),
        pl.BlockSpec((F, IN_CH), lambda i: (0, 0)),
        pl.BlockSpec((1, F), lambda i: (0, 0)),
        _d_spec,
        _d_spec,
    ],
    out_specs=[_q_spec] * (3 * NG) + [_d_spec, _d_spec],
    out_shape=[_q_shape] * (3 * NG)
              + [jax.ShapeDtypeStruct((N, 1), jnp.float32)] * 2,
)


@functools.partial(
    pl.kernel,
    out_type=jax.ShapeDtypeStruct((6 * OST, FQ), jnp.float32),
    mesh=_mesh,
    scratch_types=[
        pltpu.VMEM_SHARED((NROWS, FQ), jnp.float32),
        pltpu.VMEM((2 * GB,), jnp.int32),
        pltpu.VMEM((2, IC, B), jnp.int32),
        pltpu.VMEM((2, GB, FQ), jnp.float32),
        pltpu.VMEM((CPY, FQ), jnp.float32),
        pltpu.SemaphoreType.DMA,
        pltpu.SemaphoreType.DMA,
        pltpu.SemaphoreType.DMA,
        pltpu.SemaphoreType.DMA,
    ],
    compiler_params=pltpu.CompilerParams(use_tc_tiling_on_sc=False),
)
def _step_kernel(srcall, dst2, zq, gall, aall,
                 acc, sidx, didx, rows, vbuf, gsem0, gsem1, ssem0, ssem1):
    c = lax.axis_index("c")
    s = lax.axis_index("s")
    gsems = (gsem0, gsem1)
    ssems = (ssem0, ssem1)

    def one_pass(pp, carry):
        first = pp < 2
        q = jnp.where(first, 2 * c + pp, 4)
        ebase = jnp.where(first, s * RPT,
                          c * (EROWS // 2) + s * (RPT // 2))
        nic = jnp.where(first, NIC0, NIC2)
        obase = jnp.where(first, (2 * c + pp) * OST, (4 + c) * OST)

        pltpu.sync_copy(zq, vbuf)
        for i in range(NCPY):
            pltpu.sync_copy(vbuf, acc.at[pl.ds(s * ROWS_PT + i * CPY, CPY)])
        plsc.subcore_barrier()

        def pair(kk, carry2):
            for half in range(2):
                k = 2 * kk + half
                di = didx.at[half]

                @pl.when(k < nic)
                def _(k=k, di=di):
                    pltpu.sync_copy(
                        srcall.at[q].at[pl.ds(ebase * B + k * 2 * GB,
                                              2 * GB)], sidx)
                    pltpu.sync_copy(dst2.at[pl.ds(ebase + k * IC, IC)], di)
                    for pos in range(2):
                        @pl.when(k > 0)
                        def _(pos=pos, di=di):
                            for r in range(SPB):
                                pltpu.make_async_copy(
                                    rows.at[pos].at[pl.ds(r * B, B)],
                                    acc.at[di.at[pos * SPB + r]],
                                    ssems[pos]).wait()
                        pltpu.async_copy(
                            gall.at[sidx.at[pl.ds(pos * GB, GB)]],
                            rows.at[pos], gsems[pos])
                    for pos in range(2):
                        pltpu.make_async_copy(
                            gall.at[sidx.at[pl.ds(pos * GB, GB)]],
                            rows.at[pos], gsems[pos]).wait()
                        for r in range(SPB):
                            pltpu.async_copy(
                                rows.at[pos].at[pl.ds(r * B, B)],
                                acc.at[di.at[pos * SPB + r]],
                                ssems[pos], add=True)
            return carry2

        lax.fori_loop(0, NIC0 // 2, pair, 0)
        for pos in range(2):
            for r in range(SPB):
                pltpu.make_async_copy(
                    rows.at[pos].at[pl.ds(r * B, B)],
                    acc.at[didx.at[0].at[pos * SPB + r]], ssems[pos]).wait()
        plsc.subcore_barrier()

        for i in range(NCPY):
            pltpu.sync_copy(acc.at[pl.ds(s * ROWS_PT + i * CPY, CPY)], vbuf)
            pltpu.sync_copy(
                vbuf, aall.at[pl.ds(obase + s * ROWS_PT + i * CPY, CPY)])
        plsc.subcore_barrier()
        return carry

    lax.fori_loop(0, 3, one_pass, 0)


FB = 320
GR = NP * FQ // 128 // FB


def _epi_body(a_ref, a5_ref, g_ref, s_ref, t_ref, o_ref):
    q = pl.program_id(1)
    a = a_ref[...] + g_ref[...]
    a = jnp.where(q == NG - 1, a + a5_ref[...], a)
    o_ref[...] = s_ref[...] * a + t_ref[...]


_gq_spec = pl.BlockSpec((FB, 128), lambda i, q: (q * GR + i, 0))

_epi = pl.pallas_call(
    _epi_body,
    grid=(GR, NG),
    in_specs=[
        _gq_spec,
        pl.BlockSpec((FB, 128), lambda i, q: (NG * GR + i, 0)),
        _gq_spec,
        _gq_spec,
        _gq_spec,
    ],
    out_specs=[_gq_spec],
    out_shape=[jax.ShapeDtypeStruct((NG * GR * FB, 128), jnp.float32)],
)


def kernel(x, edge_index, W1, b1, W2, b2):
    src = edge_index[0]
    dst = edge_index[1]
    pad = E_PAD - E
    src2 = jnp.concatenate(
        [src, jnp.zeros((pad,), jnp.int32)]).reshape(EROWS, B)
    dst2 = jnp.concatenate(
        [dst, jnp.full((pad,), SENT, jnp.int32)]).reshape(EROWS, B)
    srcall = (src2.reshape(E_PAD)[None, :]
              + (NP * jnp.arange(NG, dtype=jnp.int32))[:, None])

    degp = _deg_kernel(dst2)
    d0 = degp[:N].reshape(N, 1)
    d1 = degp[NROWS:NROWS + N].reshape(N, 1)

    outs = _prep(x, W1, b1.reshape(1, IN_CH), W2, b2.reshape(1, F), d0, d1)

    zpadq = jnp.zeros((NP - N, FQ), jnp.float32)

    def stackq(qs):
        return jnp.concatenate(
            [jnp.concatenate([a, zpadq]) for a in qs])

    g0all = stackq(outs[:NG])
    t0f = stackq(outs[NG:2 * NG]).reshape(-1, 128)
    u0f = stackq(outs[2 * NG:3 * NG]).reshape(-1, 128)
    zpadd = jnp.zeros((NP - N,), jnp.float32)
    s2f = jnp.tile(jnp.repeat(
        jnp.concatenate([outs[3 * NG][:, 0], zpadd]), FQ), NG).reshape(-1, 128)
    s1f = jnp.tile(jnp.repeat(
        jnp.concatenate([outs[3 * NG + 1][:, 0], zpadd]), FQ), NG).reshape(-1, 128)

    zq = jnp.zeros((CPY, FQ), jnp.float32)
    gall = g0all
    for step in range(K):
        aall = _step_kernel(srcall, dst2, zq, gall)
        af = aall.reshape(-1, 128)
        gf = gall.reshape(-1, 128)
        if step < K - 1:
            (gfn,) = _epi(af, af, gf, s2f, t0f)
            gall = gfn.reshape(NG * NP, FQ)
        else:
            (hf,) = _epi(af, af, gf, s1f, u0f)
    hall = hf.reshape(NG, NP, FQ)[:, :N, :]
    return jnp.transpose(hall, (1, 0, 2)).reshape(N, F)


# speedup vs baseline: 1.0916x; 1.0916x over previous
import functools

import jax
import jax.numpy as jnp
from jax import lax
from jax.experimental import pallas as pl
from jax.experimental.pallas import tpu as pltpu
from jax.experimental.pallas import tpu_sc as plsc

N = 100000
E = 1600000
IN_CH = 128
F = 40
FQ = 8
NG = 5
K = 10
ALPHA = 0.1

B = 128
IC = 40
BK = 20
NT = 16
EROWS = 12800
E_PAD = EROWS * B
RPT = EROWS // NT
NIC0 = RPT // IC
NIC2 = RPT // 2 // IC
ROWS_PT = 6256
NROWS = ROWS_PT * NT
SENT = N
CPY = 368
NCPY = ROWS_PT // CPY
NP = 102400
OST = NP

DCH = 16
DNCH = 25

RB = 2000
GRID = N // RB

_mesh = plsc.VectorSubcoreMesh(core_axis_name="c", subcore_axis_name="s")


@functools.partial(
    pl.kernel,
    out_type=jax.ShapeDtypeStruct((2 * NROWS,), jnp.float32),
    mesh=_mesh,
    scratch_types=[
        pltpu.VMEM_SHARED((NROWS,), jnp.float32),
        pltpu.VMEM((DCH, B), jnp.int32),
        pltpu.VMEM((B,), jnp.float32),
        pltpu.VMEM((ROWS_PT,), jnp.float32),
        pltpu.SemaphoreType.DMA,
    ],
    compiler_params=pltpu.CompilerParams(use_tc_tiling_on_sc=False),
)
def _deg_kernel(dst2, deg_out, acc, didx, ones, zbuf, ssem):
    c = lax.axis_index("c")
    s = lax.axis_index("s")
    for i in range(B // 16):
        ones[pl.ds(i * 16, 16)] = jnp.ones((16,), jnp.float32)
    for i in range(ROWS_PT // 16):
        zbuf[pl.ds(i * 16, 16)] = jnp.zeros((16,), jnp.float32)
    pltpu.sync_copy(zbuf, acc.at[pl.ds(s * ROWS_PT, ROWS_PT)])
    plsc.subcore_barrier()

    w = c * NT + s
    r0 = w * (DCH * DNCH)

    def chunk(k, carry):
        rb = r0 + k * DCH
        pltpu.sync_copy(dst2.at[pl.ds(rb, DCH)], didx)
        for j in range(DCH):
            pltpu.async_copy(ones, acc.at[didx.at[j]], ssem, add=True)
        for j in range(DCH):
            pltpu.make_async_copy(ones, acc.at[didx.at[j]], ssem).wait()
        return carry

    lax.fori_loop(0, DNCH, chunk, 0)
    plsc.subcore_barrier()

    pltpu.sync_copy(acc.at[pl.ds(s * ROWS_PT, ROWS_PT)], zbuf)
    pltpu.sync_copy(zbuf, deg_out.at[pl.ds(c * NROWS + s * ROWS_PT, ROWS_PT)])


def _prep_body(x_ref, w1_ref, b1_ref, w2_ref, b2_ref, d0_ref, d1_ref,
               *outs):
    xb = x_ref[...]
    h = lax.dot_general(xb, w1_ref[...], (((1,), (1,)), ((), ())),
                        preferred_element_type=jnp.float32)
    h = jnp.maximum(h + b1_ref[...], 0.0)
    y = lax.dot_general(h, w2_ref[...], (((1,), (1,)), ((), ())),
                        preferred_element_type=jnp.float32)
    y = y + b2_ref[...]
    deg = d0_ref[...] + d1_ref[...] + 1.0
    dinv = lax.rsqrt(deg)
    g0 = y * dinv
    t0 = ALPHA * g0
    u0 = ALPHA * y
    for q in range(NG):
        outs[q][...] = g0[:, q * FQ:(q + 1) * FQ]
        outs[NG + q][...] = t0[:, q * FQ:(q + 1) * FQ]
        outs[2 * NG + q][...] = u0[:, q * FQ:(q + 1) * FQ]
    outs[3 * NG][...] = (1.0 - ALPHA) * dinv * dinv
    outs[3 * NG + 1][...] = (1.0 - ALPHA) * dinv


_q_spec = pl.BlockSpec((RB, FQ), lambda i: (i, 0))
_q_shape = jax.ShapeDtypeStruct((N, FQ), jnp.float32)
_d_spec = pl.BlockSpec((RB, 1), lambda i: (i, 0))

_prep = pl.pallas_call(
    _prep_body,
    grid=(GRID,),
    in_specs=[
        pl.BlockSpec((RB, IN_CH), lambda i: (i, 0)),
        pl.BlockSpec((IN_CH, IN_CH), lambda i: (0, 0)),
        pl.BlockSpec((1, IN_CH), lambda i: (0, 0)),
        pl.BlockSpec((F, IN_CH), lambda i: (0, 0)),
        pl.BlockSpec((1, F), lambda i: (0, 0)),
        _d_spec,
        _d_spec,
    ],
    out_specs=[_q_spec] * (3 * NG) + [_d_spec, _d_spec],
    out_shape=[_q_shape] * (3 * NG)
              + [jax.ShapeDtypeStruct((N, 1), jnp.float32)] * 2,
)


@functools.partial(
    pl.kernel,
    out_type=jax.ShapeDtypeStruct((6 * OST, FQ), jnp.float32),
    mesh=_mesh,
    scratch_types=[
        pltpu.VMEM_SHARED((NROWS, FQ), jnp.float32),
        pltpu.VMEM((IC, B), jnp.int32),
        pltpu.VMEM((2, IC, B), jnp.int32),
        pltpu.VMEM((2, BK, B, FQ), jnp.float32),
        pltpu.VMEM((CPY, FQ), jnp.float32),
        pltpu.SemaphoreType.DMA,
        pltpu.SemaphoreType.DMA,
        pltpu.SemaphoreType.DMA,
    ],
    compiler_params=pltpu.CompilerParams(use_tc_tiling_on_sc=False),
)
def _step_kernel(srcall, dst2, zq, gall, aall,
                 acc, sidx, didx, rows, vbuf, gsem, ssem0, ssem1):
    c = lax.axis_index("c")
    s = lax.axis_index("s")
    ssems = (ssem0, ssem1)

    def one_pass(pp, carry):
        first = pp < 2
        q = jnp.where(first, 2 * c + pp, 4)
        ebase = jnp.where(first, s * RPT,
                          jnp.where(c == 0, s * 560, 8960 + s * 240))
        nic = jnp.where(first, NIC0, jnp.where(c == 0, 14, 6))
        obase = jnp.where(first, (2 * c + pp) * OST, (4 + c) * OST)

        pltpu.sync_copy(zq, vbuf)
        for i in range(NCPY):
            pltpu.sync_copy(vbuf, acc.at[pl.ds(s * ROWS_PT + i * CPY, CPY)])
        plsc.subcore_barrier()

        def pair(kk, carry2):
            for half in range(2):
                k = 2 * kk + half
                di = didx.at[half]

                @pl.when(k < nic)
                def _(k=k, di=di):
                    rb = ebase + k * IC
                    pltpu.sync_copy(srcall.at[q].at[pl.ds(rb, IC)], sidx)
                    pltpu.sync_copy(dst2.at[pl.ds(rb, IC)], di)
                    for pos in range(2):
                        bank = rows.at[pos]
                        ssem = ssems[pos]
                        off = pos * BK

                        @pl.when(k > 0)
                        def _(bank=bank, ssem=ssem, di=di, off=off):
                            for j in range(BK):
                                pltpu.make_async_copy(
                                    bank.at[j], acc.at[di.at[off + j]],
                                    ssem).wait()
                        for j in range(BK):
                            pltpu.async_copy(gall.at[sidx.at[off + j]],
                                             bank.at[j], gsem)
                        for j in range(BK):
                            pltpu.make_async_copy(
                                gall.at[sidx.at[off + j]],
                                bank.at[j], gsem).wait()
                        for j in range(BK):
                            pltpu.async_copy(bank.at[j],
                                             acc.at[di.at[off + j]],
                                             ssem, add=True)
            return carry2

        lax.fori_loop(0, NIC0 // 2, pair, 0)
        for pos in range(2):
            for j in range(BK):
                pltpu.make_async_copy(
                    rows.at[pos].at[j],
                    acc.at[didx.at[0].at[pos * BK + j]], ssems[pos]).wait()
        plsc.subcore_barrier()

        for i in range(NCPY):
            pltpu.sync_copy(acc.at[pl.ds(s * ROWS_PT + i * CPY, CPY)], vbuf)
            pltpu.sync_copy(
                vbuf, aall.at[pl.ds(obase + s * ROWS_PT + i * CPY, CPY)])
        plsc.subcore_barrier()
        return carry

    lax.fori_loop(0, 3, one_pass, 0)


FB = 320
GR = NP * FQ // 128 // FB


def _epi_body(a_ref, a5_ref, g_ref, s_ref, t_ref, o_ref):
    q = pl.program_id(1)
    a = a_ref[...] + g_ref[...]
    a = jnp.where(q == NG - 1, a + a5_ref[...], a)
    o_ref[...] = s_ref[...] * a + t_ref[...]


_gq_spec = pl.BlockSpec((FB, 128), lambda i, q: (q * GR + i, 0))

_epi = pl.pallas_call(
    _epi_body,
    grid=(GR, NG),
    in_specs=[
        _gq_spec,
        pl.BlockSpec((FB, 128), lambda i, q: (NG * GR + i, 0)),
        _gq_spec,
        _gq_spec,
        _gq_spec,
    ],
    out_specs=[_gq_spec],
    out_shape=[jax.ShapeDtypeStruct((NG * GR * FB, 128), jnp.float32)],
)


def kernel(x, edge_index, W1, b1, W2, b2):
    src = edge_index[0]
    dst = edge_index[1]
    pad = E_PAD - E
    src2 = jnp.concatenate(
        [src, jnp.zeros((pad,), jnp.int32)]).reshape(EROWS, B)
    dst2 = jnp.concatenate(
        [dst, jnp.full((pad,), SENT, jnp.int32)]).reshape(EROWS, B)
    srcall = src2[None, :, :] + (NP * jnp.arange(NG, dtype=jnp.int32)
                                 )[:, None, None]

    degp = _deg_kernel(dst2)
    d0 = degp[:N].reshape(N, 1)
    d1 = degp[NROWS:NROWS + N].reshape(N, 1)

    outs = _prep(x, W1, b1.reshape(1, IN_CH), W2, b2.reshape(1, F), d0, d1)

    zpadq = jnp.zeros((NP - N, FQ), jnp.float32)

    def stackq(qs):
        return jnp.concatenate(
            [jnp.concatenate([a, zpadq]) for a in qs])

    g0all = stackq(outs[:NG])
    t0f = stackq(outs[NG:2 * NG]).reshape(-1, 128)
    u0f = stackq(outs[2 * NG:3 * NG]).reshape(-1, 128)
    zpadd = jnp.zeros((NP - N,), jnp.float32)
    s2f = jnp.tile(jnp.repeat(
        jnp.concatenate([outs[3 * NG][:, 0], zpadd]), FQ), NG).reshape(-1, 128)
    s1f = jnp.tile(jnp.repeat(
        jnp.concatenate([outs[3 * NG + 1][:, 0], zpadd]), FQ), NG).reshape(-1, 128)

    zq = jnp.zeros((CPY, FQ), jnp.float32)
    gall = g0all
    for step in range(K):
        aall = _step_kernel(srcall, dst2, zq, gall)
        af = aall.reshape(-1, 128)
        gf = gall.reshape(-1, 128)
        if step < K - 1:
            (gfn,) = _epi(af, af, gf, s2f, t0f)
            gall = gfn.reshape(NG * NP, FQ)
        else:
            (hf,) = _epi(af, af, gf, s1f, u0f)
    hall = hf.reshape(NG, NP, FQ)[:, :N, :]
    return jnp.transpose(hall, (1, 0, 2)).reshape(N, F)


# speedup vs baseline: 1.0931x; 1.0014x over previous
import functools

import jax
import jax.numpy as jnp
from jax import lax
from jax.experimental import pallas as pl
from jax.experimental.pallas import tpu as pltpu
from jax.experimental.pallas import tpu_sc as plsc

N = 100000
E = 1600000
IN_CH = 128
F = 40
FQ = 8
NG = 5
K = 10
ALPHA = 0.1

B = 128
IC = 40
BK = 20
NT = 16
EROWS = 12800
E_PAD = EROWS * B
RPT = EROWS // NT
NIC0 = RPT // IC
NIC2 = RPT // 2 // IC
ROWS_PT = 6256
NROWS = ROWS_PT * NT
SENT = N
CPY = 368
NCPY = ROWS_PT // CPY
NP = 102400
OST = NP

DCH = 16
DNCH = 25

RB = 2000
GRID = N // RB

_mesh = plsc.VectorSubcoreMesh(core_axis_name="c", subcore_axis_name="s")


@functools.partial(
    pl.kernel,
    out_type=jax.ShapeDtypeStruct((2 * NROWS,), jnp.float32),
    mesh=_mesh,
    scratch_types=[
        pltpu.VMEM_SHARED((NROWS,), jnp.float32),
        pltpu.VMEM((DCH, B), jnp.int32),
        pltpu.VMEM((B,), jnp.float32),
        pltpu.VMEM((ROWS_PT,), jnp.float32),
        pltpu.SemaphoreType.DMA,
    ],
    compiler_params=pltpu.CompilerParams(use_tc_tiling_on_sc=False),
)
def _deg_kernel(dst2, deg_out, acc, didx, ones, zbuf, ssem):
    c = lax.axis_index("c")
    s = lax.axis_index("s")
    for i in range(B // 16):
        ones[pl.ds(i * 16, 16)] = jnp.ones((16,), jnp.float32)
    for i in range(ROWS_PT // 16):
        zbuf[pl.ds(i * 16, 16)] = jnp.zeros((16,), jnp.float32)
    pltpu.sync_copy(zbuf, acc.at[pl.ds(s * ROWS_PT, ROWS_PT)])
    plsc.subcore_barrier()

    w = c * NT + s
    r0 = w * (DCH * DNCH)

    def chunk(k, carry):
        rb = r0 + k * DCH
        pltpu.sync_copy(dst2.at[pl.ds(rb, DCH)], didx)
        for j in range(DCH):
            pltpu.async_copy(ones, acc.at[didx.at[j]], ssem, add=True)
        for j in range(DCH):
            pltpu.make_async_copy(ones, acc.at[didx.at[j]], ssem).wait()
        return carry

    lax.fori_loop(0, DNCH, chunk, 0)
    plsc.subcore_barrier()

    pltpu.sync_copy(acc.at[pl.ds(s * ROWS_PT, ROWS_PT)], zbuf)
    pltpu.sync_copy(zbuf, deg_out.at[pl.ds(c * NROWS + s * ROWS_PT, ROWS_PT)])


def _prep_body(x_ref, w1_ref, b1_ref, w2_ref, b2_ref, d0_ref, d1_ref,
               *outs):
    xb = x_ref[...]
    h = lax.dot_general(xb, w1_ref[...], (((1,), (1,)), ((), ())),
                        preferred_element_type=jnp.float32)
    h = jnp.maximum(h + b1_ref[...], 0.0)
    y = lax.dot_general(h, w2_ref[...], (((1,), (1,)), ((), ())),
                        preferred_element_type=jnp.float32)
    y = y + b2_ref[...]
    deg = d0_ref[...] + d1_ref[...] + 1.0
    dinv = lax.rsqrt(deg)
    g0 = y * dinv
    t0 = ALPHA * g0
    u0 = ALPHA * y
    for q in range(NG):
        outs[q][...] = g0[:, q * FQ:(q + 1) * FQ]
        outs[NG + q][...] = t0[:, q * FQ:(q + 1) * FQ]
        outs[2 * NG + q][...] = u0[:, q * FQ:(q + 1) * FQ]
    outs[3 * NG][...] = (1.0 - ALPHA) * dinv * dinv
    outs[3 * NG + 1][...] = (1.0 - ALPHA) * dinv


_q_spec = pl.BlockSpec((RB, FQ), lambda i: (i, 0))
_q_shape = jax.ShapeDtypeStruct((N, FQ), jnp.float32)
_d_spec = pl.BlockSpec((RB, 1), lambda i: (i, 0))

_prep = pl.pallas_call(
    _prep_body,
    grid=(GRID,),
    in_specs=[
        pl.BlockSpec((RB, IN_CH), lambda i: (i, 0)),
        pl.BlockSpec((IN_CH, IN_CH), lambda i: (0, 0)),
        pl.BlockSpec((1, IN_CH), lambda i: (0, 0)),
        pl.BlockSpec((F, IN_CH), lambda i: (0, 0)),
        pl.BlockSpec((1, F), lambda i: (0, 0)),
        _d_spec,
        _d_spec,
    ],
    out_specs=[_q_spec] * (3 * NG) + [_d_spec, _d_spec],
    out_shape=[_q_shape] * (3 * NG)
              + [jax.ShapeDtypeStruct((N, 1), jnp.float32)] * 2,
)


@functools.partial(
    pl.kernel,
    out_type=jax.ShapeDtypeStruct((6 * OST, FQ), jnp.float32),
    mesh=_mesh,
    scratch_types=[
        pltpu.VMEM_SHARED((NROWS, FQ), jnp.float32),
        pltpu.VMEM((IC, B), jnp.int32),
        pltpu.VMEM((2, IC, B), jnp.int32),
        pltpu.VMEM((2, BK, B, FQ), jnp.float32),
        pltpu.VMEM((CPY, FQ), jnp.float32),
        pltpu.SemaphoreType.DMA,
        pltpu.SemaphoreType.DMA,
        pltpu.SemaphoreType.DMA,
        pltpu.SemaphoreType.DMA,
    ],
    compiler_params=pltpu.CompilerParams(use_tc_tiling_on_sc=False),
)
def _step_kernel(srcall, dst2, zq, gall, aall,
                 acc, sidx, didx, rows, vbuf, gsem, ssem0, ssem1, zsem):
    c = lax.axis_index("c")
    s = lax.axis_index("s")
    ssems = (ssem0, ssem1)

    def one_pass(pp, carry):
        first = pp < 2
        q = jnp.where(first, 2 * c + pp, 4)
        ebase = jnp.where(first, s * RPT,
                          jnp.where(c == 0, s * 560, 8960 + s * 240))
        nic = jnp.where(first, NIC0, jnp.where(c == 0, 14, 6))
        obase = jnp.where(first, (2 * c + pp) * OST, (4 + c) * OST)

        pltpu.sync_copy(zq, vbuf)
        for i in range(NCPY):
            pltpu.async_copy(
                vbuf, acc.at[pl.ds(s * ROWS_PT + i * CPY, CPY)], zsem)
        for i in range(NCPY):
            pltpu.make_async_copy(
                vbuf, acc.at[pl.ds(s * ROWS_PT + i * CPY, CPY)], zsem).wait()
        plsc.subcore_barrier()

        def pair(kk, carry2):
            for half in range(2):
                k = 2 * kk + half
                di = didx.at[half]

                @pl.when(k < nic)
                def _(k=k, di=di):
                    rb = ebase + k * IC
                    pltpu.sync_copy(srcall.at[q].at[pl.ds(rb, IC)], sidx)
                    pltpu.sync_copy(dst2.at[pl.ds(rb, IC)], di)
                    for pos in range(2):
                        bank = rows.at[pos]
                        ssem = ssems[pos]
                        off = pos * BK

                        @pl.when(k > 0)
                        def _(bank=bank, ssem=ssem, di=di, off=off):
                            for j in range(BK):
                                pltpu.make_async_copy(
                                    bank.at[j], acc.at[di.at[off + j]],
                                    ssem).wait()
                        for j in range(BK):
                            pltpu.async_copy(gall.at[sidx.at[off + j]],
                                             bank.at[j], gsem)
                        for j in range(BK):
                            pltpu.make_async_copy(
                                gall.at[sidx.at[off + j]],
                                bank.at[j], gsem).wait()
                        for j in range(BK):
                            pltpu.async_copy(bank.at[j],
                                             acc.at[di.at[off + j]],
                                             ssem, add=True)
            return carry2

        lax.fori_loop(0, NIC0 // 2, pair, 0)
        for pos in range(2):
            for j in range(BK):
                pltpu.make_async_copy(
                    rows.at[pos].at[j],
                    acc.at[didx.at[0].at[pos * BK + j]], ssems[pos]).wait()
        plsc.subcore_barrier()

        for i in range(NCPY):
            pltpu.sync_copy(acc.at[pl.ds(s * ROWS_PT + i * CPY, CPY)], vbuf)
            pltpu.sync_copy(
                vbuf, aall.at[pl.ds(obase + s * ROWS_PT + i * CPY, CPY)])
        plsc.subcore_barrier()
        return carry

    lax.fori_loop(0, 3, one_pass, 0)


FB = 320
GR = NP * FQ // 128 // FB


def _epi_body(a_ref, a5_ref, g_ref, s_ref, t_ref, o_ref):
    q = pl.program_id(1)
    a = a_ref[...] + g_ref[...]
    a = jnp.where(q == NG - 1, a + a5_ref[...], a)
    o_ref[...] = s_ref[...] * a + t_ref[...]


_gq_spec = pl.BlockSpec((FB, 128), lambda i, q: (q * GR + i, 0))

_epi = pl.pallas_call(
    _epi_body,
    grid=(GR, NG),
    in_specs=[
        _gq_spec,
        pl.BlockSpec((FB, 128), lambda i, q: (NG * GR + i, 0)),
        _gq_spec,
        _gq_spec,
        _gq_spec,
    ],
    out_specs=[_gq_spec],
    out_shape=[jax.ShapeDtypeStruct((NG * GR * FB, 128), jnp.float32)],
)


def kernel(x, edge_index, W1, b1, W2, b2):
    src = edge_index[0]
    dst = edge_index[1]
    pad = E_PAD - E
    src2 = jnp.concatenate(
        [src, jnp.zeros((pad,), jnp.int32)]).reshape(EROWS, B)
    dst2 = jnp.concatenate(
        [dst, jnp.full((pad,), SENT, jnp.int32)]).reshape(EROWS, B)
    srcall = src2[None, :, :] + (NP * jnp.arange(NG, dtype=jnp.int32)
                                 )[:, None, None]

    degp = _deg_kernel(dst2)
    d0 = degp[:N].reshape(N, 1)
    d1 = degp[NROWS:NROWS + N].reshape(N, 1)

    outs = _prep(x, W1, b1.reshape(1, IN_CH), W2, b2.reshape(1, F), d0, d1)

    zpadq = jnp.zeros((NP - N, FQ), jnp.float32)

    def stackq(qs):
        return jnp.concatenate(
            [jnp.concatenate([a, zpadq]) for a in qs])

    g0all = stackq(outs[:NG])
    t0f = stackq(outs[NG:2 * NG]).reshape(-1, 128)
    u0f = stackq(outs[2 * NG:3 * NG]).reshape(-1, 128)
    zpadd = jnp.zeros((NP - N,), jnp.float32)
    s2f = jnp.tile(jnp.repeat(
        jnp.concatenate([outs[3 * NG][:, 0], zpadd]), FQ), NG).reshape(-1, 128)
    s1f = jnp.tile(jnp.repeat(
        jnp.concatenate([outs[3 * NG + 1][:, 0], zpadd]), FQ), NG).reshape(-1, 128)

    zq = jnp.zeros((CPY, FQ), jnp.float32)
    gall = g0all
    for step in range(K):
        aall = _step_kernel(srcall, dst2, zq, gall)
        af = aall.reshape(-1, 128)
        gf = gall.reshape(-1, 128)
        if step < K - 1:
            (gfn,) = _epi(af, af, gf, s2f, t0f)
            gall = gfn.reshape(NG * NP, FQ)
        else:
            (hf,) = _epi(af, af, gf, s1f, u0f)
    hall = hf.reshape(NG, NP, FQ)[:, :N, :]
    return jnp.transpose(hall, (1, 0, 2)).reshape(N, F)
